# baseline (speedup 1.0000x reference)
import functools

import jax
import jax.numpy as jnp
from jax import lax
from jax.experimental import pallas as pl
from jax.experimental.pallas import tpu as pltpu
from jax.experimental.pallas import tpu_sc as plsc

NC = 2
NS = 16
NW = NC * NS
C = 80
NBUF = 3
CH0 = 156
CH1 = 96
SHIFT = 14

_mesh = plsc.VectorSubcoreMesh(
    core_axis_name="c", subcore_axis_name="s", num_cores=NC, num_subcores=NS
)
_sc_params = pltpu.CompilerParams(
    needs_layout_passes=False, use_tc_tiling_on_sc=False
)


def _make_deg_kernel(n_pad, ch_max):
    @functools.partial(
        pl.kernel,
        out_type=jax.ShapeDtypeStruct((NW, n_pad), jnp.float32),
        mesh=_mesh,
        compiler_params=_sc_params,
        scratch_types=[
            pltpu.VMEM((n_pad,), jnp.float32),
            pltpu.VMEM((ch_max, C), jnp.int32),
        ],
    )
    def deg_kernel(pk_hbm, out_hbm, hist, pkbuf):
        c = lax.axis_index("c")
        s = lax.axis_index("s")
        wid = s * NC + c

        def zero_body(i, _):
            hist[pl.ds(i * 16, 16)] = jnp.zeros((16,), jnp.float32)
            return 0

        lax.fori_loop(0, n_pad // 16, zero_body, 0)
        pltpu.sync_copy(pk_hbm.at[wid], pkbuf)

        ones = jnp.full((16,), 1.0, jnp.float32)

        def body(k, _):
            for j in range(C // 16):
                v = pkbuf[k, pl.ds(j * 16, 16)]
                dst16 = lax.shift_right_logical(v, SHIFT)
                plsc.addupdate_scatter(hist, [dst16], ones)
            return 0

        lax.fori_loop(0, ch_max, body, 0)
        pltpu.sync_copy(hist, out_hbm.at[wid])

    return deg_kernel


def _make_prop_kernel(n, n_pad, ch_max, d, rows_per):
    @functools.partial(
        pl.kernel,
        out_type=jax.ShapeDtypeStruct((NC, n, d), jnp.float32),
        mesh=_mesh,
        compiler_params=_sc_params,
        scratch_types=[
            pltpu.VMEM_SHARED((n_pad, d), jnp.float32),
            pltpu.VMEM((ch_max, C), jnp.int32),
            pltpu.VMEM((NBUF, C), jnp.int32),
            pltpu.VMEM((NBUF, C), jnp.int32),
            tuple(pltpu.VMEM((C, d), jnp.float32) for _ in range(NBUF)),
            pltpu.SemaphoreType.DMA((NBUF,)),
        ],
    )
    def prop_kernel(p_hbm, pk_hbm, out_hbm, acc, pkbuf, sring, dring, rows,
                    gsem):
        c = lax.axis_index("c")
        s = lax.axis_index("s")
        wid = s * NC + c
        nch = lax.select(c == 0, CH0, CH1)

        tail = n - NS * rows_per
        pltpu.sync_copy(
            p_hbm.at[pl.ds(s * rows_per, rows_per)],
            acc.at[pl.ds(s * rows_per, rows_per)],
        )
        @pl.when(s == 0)
        def _():
            pltpu.sync_copy(
                p_hbm.at[pl.ds(NS * rows_per, tail)],
                acc.at[pl.ds(NS * rows_per, tail)],
            )
        pltpu.sync_copy(pk_hbm.at[wid], pkbuf)

        def unpack(k, slot):
            for j in range(C // 16):
                v = pkbuf[k, pl.ds(j * 16, 16)]
                sring[slot, pl.ds(j * 16, 16)] = v & ((1 << SHIFT) - 1)
                dring[slot, pl.ds(j * 16, 16)] = lax.shift_right_logical(v, SHIFT)

        def fire_gather(k, b):
            pltpu.async_copy(p_hbm.at[sring.at[b]], rows[b], gsem.at[b])

        def wait_gather(k, b):
            pltpu.make_async_copy(p_hbm.at[sring.at[b]], rows[b], gsem.at[b]).wait()

        for b in range(NBUF - 1):
            unpack(b, b)
            fire_gather(b, b)
        plsc.subcore_barrier()

        def body(g, _):
            for b in range(NBUF):
                k = g * NBUF + b
                wait_gather(k, b)

                @pl.when(k + NBUF - 1 < nch)
                def _():
                    unpack(k + NBUF - 1, (b + NBUF - 1) % NBUF)
                    fire_gather(k + NBUF - 1, (b + NBUF - 1) % NBUF)

                pltpu.sync_copy(rows[b], acc.at[dring.at[b]], add=True)
            return 0

        lax.fori_loop(0, nch // NBUF, body, 0)
        plsc.subcore_barrier()
        pltpu.sync_copy(
            acc.at[pl.ds(s * rows_per, rows_per)],
            out_hbm.at[c, pl.ds(s * rows_per, rows_per)],
        )
        @pl.when(s == 0)
        def _():
            pltpu.sync_copy(
                acc.at[pl.ds(NS * rows_per, tail)],
                out_hbm.at[c, pl.ds(NS * rows_per, tail)],
            )

    return prop_kernel


def _deg_reduce_body(hist_ref, dis_ref):
    deg = 1.0 + jnp.sum(hist_ref[...], axis=0, keepdims=True)
    dis_ref[...] = lax.rsqrt(deg)


def _t0_body(x_ref, dis_ref, w_ref, p_ref):
    q = jnp.dot(x_ref[...], w_ref[...], preferred_element_type=jnp.float32)
    p_ref[...] = q * dis_ref[...]


def _t_mid_body(acc_ref, p_ref, dis_ref, b_ref, w_ref, o_ref):
    dis = dis_ref[...]
    a = acc_ref[...]
    h = (a[0] + a[1] - p_ref[...]) * dis + b_ref[...]
    h = jnp.maximum(h, 0.0)
    o_ref[...] = jnp.dot(h, w_ref[...], preferred_element_type=jnp.float32) * dis


def _t_final_body(acc_ref, p_ref, dis_ref, b_ref, o_ref):
    a = acc_ref[...]
    o_ref[...] = (a[0] + a[1] - p_ref[...]) * dis_ref[...] + b_ref[...]


def kernel(x, edge_index, batch, W1, b1, W2, b2, W3, b3):
    n, d_in = x.shape
    e = edge_index.shape[1]
    d_hid = W1.shape[1]
    n_cls = W3.shape[1]

    ch_max = max(CH0, CH1)
    n_pad = -(-(n + 1) // NS) * NS
    rows_per = (n // NS) // 8 * 8

    src = edge_index[0]
    dst = edge_index[1]
    pad_val = jnp.int32(n << SHIFT)
    packed = src + (dst << SHIFT)

    cap1 = NS * CH1 * C
    cap0 = NS * CH0 * C
    r1 = cap1
    pk1 = packed[:r1].reshape(NS, CH1, C)
    pk0 = jnp.pad(packed[r1:], (0, cap0 - (e - r1)),
                  constant_values=pad_val).reshape(NS, CH0, C)
    pk0 = jnp.pad(pk0, ((0, 0), (0, ch_max - CH0), (0, 0)),
                  constant_values=pad_val)
    pk1 = jnp.pad(pk1, ((0, 0), (0, ch_max - CH1), (0, 0)),
                  constant_values=pad_val)
    pk = jnp.stack([pk0, pk1], axis=1).reshape(NW, ch_max, C)

    hist = _make_deg_kernel(n_pad, ch_max)(pk)

    bn = 400
    grid = n // bn
    f32 = jnp.float32

    dis_full = pl.pallas_call(
        _deg_reduce_body,
        grid=(1,),
        in_specs=[pl.BlockSpec((NW, n_pad), lambda j: (0, 0))],
        out_specs=pl.BlockSpec((1, n_pad), lambda j: (0, 0)),
        out_shape=jax.ShapeDtypeStruct((1, n_pad), f32),
    )(hist)
    dis = dis_full[0, :n].reshape(n, 1)

    p1 = pl.pallas_call(
        _t0_body,
        grid=(grid,),
        in_specs=[
            pl.BlockSpec((bn, d_in), lambda j: (j, 0)),
            pl.BlockSpec((bn, 1), lambda j: (j, 0)),
            pl.BlockSpec((d_in, d_hid), lambda j: (0, 0)),
        ],
        out_specs=pl.BlockSpec((bn, d_hid), lambda j: (j, 0)),
        out_shape=jax.ShapeDtypeStruct((n, d_hid), f32),
    )(x, dis, W1)

    prop_h = _make_prop_kernel(n, n_pad, ch_max, d_hid, rows_per)
    prop_c = _make_prop_kernel(n, n_pad, ch_max, n_cls, rows_per)

    def t_mid(acc, p, dis, b, w, d_out):
        return pl.pallas_call(
            _t_mid_body,
            grid=(grid,),
            in_specs=[
                pl.BlockSpec((NC, bn, d_hid), lambda j: (0, j, 0)),
                pl.BlockSpec((bn, d_hid), lambda j: (j, 0)),
                pl.BlockSpec((bn, 1), lambda j: (j, 0)),
                pl.BlockSpec((1, d_hid), lambda j: (0, 0)),
                pl.BlockSpec((d_hid, d_out), lambda j: (0, 0)),
            ],
            out_specs=pl.BlockSpec((bn, d_out), lambda j: (j, 0)),
            out_shape=jax.ShapeDtypeStruct((n, d_out), f32),
        )(acc, p, dis, b.reshape(1, -1), w)

    a1 = prop_h(p1, pk)
    p2 = t_mid(a1, p1, dis, b1, W2, d_hid)
    a2 = prop_h(p2, pk)
    p3 = t_mid(a2, p2, dis, b2, W3, n_cls)
    a3 = prop_c(p3, pk)

    out = pl.pallas_call(
        _t_final_body,
        grid=(grid,),
        in_specs=[
            pl.BlockSpec((NC, bn, n_cls), lambda j: (0, j, 0)),
            pl.BlockSpec((bn, n_cls), lambda j: (j, 0)),
            pl.BlockSpec((bn, 1), lambda j: (j, 0)),
            pl.BlockSpec((1, n_cls), lambda j: (0, 0)),
        ],
        out_specs=pl.BlockSpec((bn, n_cls), lambda j: (j, 0)),
        out_shape=jax.ShapeDtypeStruct((n, n_cls), f32),
    )(a3, p3, dis, b3.reshape(1, -1))
    return out


# speedup vs baseline: 1.0621x; 1.0621x over previous
import functools

import jax
import jax.numpy as jnp
from jax import lax
from jax.experimental import pallas as pl
from jax.experimental.pallas import tpu as pltpu
from jax.experimental.pallas import tpu_sc as plsc

NC = 2
NS = 16
NW = NC * NS
C = 80
NBUF = 3
CH0 = 96
CH1 = 156
SHIFT = 14

_mesh = plsc.VectorSubcoreMesh(
    core_axis_name="c", subcore_axis_name="s", num_cores=NC, num_subcores=NS
)
_sc_params = pltpu.CompilerParams(
    needs_layout_passes=False, use_tc_tiling_on_sc=False
)


def _make_deg_kernel(n_pad, ch_max):
    @functools.partial(
        pl.kernel,
        out_type=jax.ShapeDtypeStruct((NW, n_pad), jnp.float32),
        mesh=_mesh,
        compiler_params=_sc_params,
        scratch_types=[
            pltpu.VMEM((n_pad,), jnp.float32),
            pltpu.VMEM((ch_max, C), jnp.int32),
        ],
    )
    def deg_kernel(pk_hbm, out_hbm, hist, pkbuf):
        c = lax.axis_index("c")
        s = lax.axis_index("s")
        wid = s * NC + c

        def zero_body(i, _):
            hist[pl.ds(i * 16, 16)] = jnp.zeros((16,), jnp.float32)
            return 0

        lax.fori_loop(0, n_pad // 16, zero_body, 0)
        pltpu.sync_copy(pk_hbm.at[wid], pkbuf)

        ones = jnp.full((16,), 1.0, jnp.float32)

        def body(k, _):
            for j in range(C // 16):
                v = pkbuf[k, pl.ds(j * 16, 16)]
                dst16 = lax.shift_right_logical(v, SHIFT)
                plsc.addupdate_scatter(hist, [dst16], ones)
            return 0

        lax.fori_loop(0, ch_max, body, 0)
        pltpu.sync_copy(hist, out_hbm.at[wid])

    return deg_kernel


def _make_prop_kernel(n, n_pad, ch_max, d, rows_per):
    @functools.partial(
        pl.kernel,
        out_type=jax.ShapeDtypeStruct((NC, n, d), jnp.float32),
        mesh=_mesh,
        compiler_params=_sc_params,
        scratch_types=[
            pltpu.VMEM_SHARED((n_pad, d), jnp.float32),
            pltpu.VMEM((ch_max, C), jnp.int32),
            pltpu.VMEM((NBUF, C), jnp.int32),
            pltpu.VMEM((NBUF, C), jnp.int32),
            tuple(pltpu.VMEM((C, d), jnp.float32) for _ in range(NBUF)),
            pltpu.SemaphoreType.DMA((NBUF,)),
        ],
    )
    def prop_kernel(p_hbm, pk_hbm, out_hbm, acc, pkbuf, sring, dring, rows,
                    gsem):
        c = lax.axis_index("c")
        s = lax.axis_index("s")
        wid = s * NC + c
        nch = lax.select(c == 0, CH0, CH1)

        tail = n - NS * rows_per
        pltpu.sync_copy(
            p_hbm.at[pl.ds(s * rows_per, rows_per)],
            acc.at[pl.ds(s * rows_per, rows_per)],
        )
        @pl.when(s == 0)
        def _():
            pltpu.sync_copy(
                p_hbm.at[pl.ds(NS * rows_per, tail)],
                acc.at[pl.ds(NS * rows_per, tail)],
            )
        pltpu.sync_copy(pk_hbm.at[wid], pkbuf)

        def unpack(k, slot):
            for j in range(C // 16):
                v = pkbuf[k, pl.ds(j * 16, 16)]
                sring[slot, pl.ds(j * 16, 16)] = v & ((1 << SHIFT) - 1)
                dring[slot, pl.ds(j * 16, 16)] = lax.shift_right_logical(v, SHIFT)

        def fire_gather(k, b):
            pltpu.async_copy(p_hbm.at[sring.at[b]], rows[b], gsem.at[b])

        def wait_gather(k, b):
            pltpu.make_async_copy(p_hbm.at[sring.at[b]], rows[b], gsem.at[b]).wait()

        for b in range(NBUF - 1):
            unpack(b, b)
            fire_gather(b, b)
        plsc.subcore_barrier()

        def body(g, _):
            for b in range(NBUF):
                k = g * NBUF + b
                wait_gather(k, b)

                @pl.when(k + NBUF - 1 < nch)
                def _():
                    unpack(k + NBUF - 1, (b + NBUF - 1) % NBUF)
                    fire_gather(k + NBUF - 1, (b + NBUF - 1) % NBUF)

                pltpu.sync_copy(rows[b], acc.at[dring.at[b]], add=True)
            return 0

        lax.fori_loop(0, nch // NBUF, body, 0)
        plsc.subcore_barrier()
        pltpu.sync_copy(
            acc.at[pl.ds(s * rows_per, rows_per)],
            out_hbm.at[c, pl.ds(s * rows_per, rows_per)],
        )
        @pl.when(s == 0)
        def _():
            pltpu.sync_copy(
                acc.at[pl.ds(NS * rows_per, tail)],
                out_hbm.at[c, pl.ds(NS * rows_per, tail)],
            )

    return prop_kernel


def _deg_reduce_body(hist_ref, dis_ref):
    deg = 1.0 + jnp.sum(hist_ref[...], axis=0, keepdims=True)
    dis_ref[...] = lax.rsqrt(deg)


def _t0_body(x_ref, dis_ref, w_ref, p_ref):
    q = jnp.dot(x_ref[...], w_ref[...], preferred_element_type=jnp.float32)
    p_ref[...] = q * dis_ref[...]


def _t_mid_body(acc_ref, p_ref, dis_ref, b_ref, w_ref, o_ref):
    dis = dis_ref[...]
    a = acc_ref[...]
    h = (a[0] + a[1] - p_ref[...]) * dis + b_ref[...]
    h = jnp.maximum(h, 0.0)
    o_ref[...] = jnp.dot(h, w_ref[...], preferred_element_type=jnp.float32) * dis


def _t_final_body(acc_ref, p_ref, dis_ref, b_ref, o_ref):
    a = acc_ref[...]
    o_ref[...] = (a[0] + a[1] - p_ref[...]) * dis_ref[...] + b_ref[...]


def kernel(x, edge_index, batch, W1, b1, W2, b2, W3, b3):
    n, d_in = x.shape
    e = edge_index.shape[1]
    d_hid = W1.shape[1]
    n_cls = W3.shape[1]

    ch_max = max(CH0, CH1)
    n_pad = -(-(n + 1) // NS) * NS
    rows_per = (n // NS) // 8 * 8

    src = edge_index[0]
    dst = edge_index[1]
    pad_val = jnp.int32(n << SHIFT)
    packed = src + (dst << SHIFT)

    cap1 = NS * CH1 * C
    cap0 = NS * CH0 * C
    r1 = cap1
    pk1 = packed[:r1].reshape(NS, CH1, C)
    pk0 = jnp.pad(packed[r1:], (0, cap0 - (e - r1)),
                  constant_values=pad_val).reshape(NS, CH0, C)
    pk0 = jnp.pad(pk0, ((0, 0), (0, ch_max - CH0), (0, 0)),
                  constant_values=pad_val)
    pk1 = jnp.pad(pk1, ((0, 0), (0, ch_max - CH1), (0, 0)),
                  constant_values=pad_val)
    pk = jnp.stack([pk0, pk1], axis=1).reshape(NW, ch_max, C)

    hist = _make_deg_kernel(n_pad, ch_max)(pk)

    bn = 400
    grid = n // bn
    f32 = jnp.float32

    dis_full = pl.pallas_call(
        _deg_reduce_body,
        grid=(1,),
        in_specs=[pl.BlockSpec((NW, n_pad), lambda j: (0, 0))],
        out_specs=pl.BlockSpec((1, n_pad), lambda j: (0, 0)),
        out_shape=jax.ShapeDtypeStruct((1, n_pad), f32),
    )(hist)
    dis = dis_full[0, :n].reshape(n, 1)

    p1 = pl.pallas_call(
        _t0_body,
        grid=(grid,),
        in_specs=[
            pl.BlockSpec((bn, d_in), lambda j: (j, 0)),
            pl.BlockSpec((bn, 1), lambda j: (j, 0)),
            pl.BlockSpec((d_in, d_hid), lambda j: (0, 0)),
        ],
        out_specs=pl.BlockSpec((bn, d_hid), lambda j: (j, 0)),
        out_shape=jax.ShapeDtypeStruct((n, d_hid), f32),
    )(x, dis, W1)

    prop_h = _make_prop_kernel(n, n_pad, ch_max, d_hid, rows_per)
    prop_c = _make_prop_kernel(n, n_pad, ch_max, n_cls, rows_per)

    def t_mid(acc, p, dis, b, w, d_out):
        return pl.pallas_call(
            _t_mid_body,
            grid=(grid,),
            in_specs=[
                pl.BlockSpec((NC, bn, d_hid), lambda j: (0, j, 0)),
                pl.BlockSpec((bn, d_hid), lambda j: (j, 0)),
                pl.BlockSpec((bn, 1), lambda j: (j, 0)),
                pl.BlockSpec((1, d_hid), lambda j: (0, 0)),
                pl.BlockSpec((d_hid, d_out), lambda j: (0, 0)),
            ],
            out_specs=pl.BlockSpec((bn, d_out), lambda j: (j, 0)),
            out_shape=jax.ShapeDtypeStruct((n, d_out), f32),
        )(acc, p, dis, b.reshape(1, -1), w)

    a1 = prop_h(p1, pk)
    p2 = t_mid(a1, p1, dis, b1, W2, d_hid)
    a2 = prop_h(p2, pk)
    p3 = t_mid(a2, p2, dis, b2, W3, n_cls)
    a3 = prop_c(p3, pk)

    out = pl.pallas_call(
        _t_final_body,
        grid=(grid,),
        in_specs=[
            pl.BlockSpec((NC, bn, n_cls), lambda j: (0, j, 0)),
            pl.BlockSpec((bn, n_cls), lambda j: (j, 0)),
            pl.BlockSpec((bn, 1), lambda j: (j, 0)),
            pl.BlockSpec((1, n_cls), lambda j: (0, 0)),
        ],
        out_specs=pl.BlockSpec((bn, n_cls), lambda j: (j, 0)),
        out_shape=jax.ShapeDtypeStruct((n, n_cls), f32),
    )(a3, p3, dis, b3.reshape(1, -1))
    return out


# speedup vs baseline: 1.0858x; 1.0223x over previous
import functools

import jax
import jax.numpy as jnp
from jax import lax
from jax.experimental import pallas as pl
from jax.experimental.pallas import tpu as pltpu
from jax.experimental.pallas import tpu_sc as plsc

NC = 2
NS = 16
NW = NC * NS
C = 80
NBUF = 3
CH0 = 96
CH1 = 156
SHIFT = 14

_mesh = plsc.VectorSubcoreMesh(
    core_axis_name="c", subcore_axis_name="s", num_cores=NC, num_subcores=NS
)
_sc_params = pltpu.CompilerParams(
    needs_layout_passes=False, use_tc_tiling_on_sc=False
)


def _make_deg_kernel(n_pad, ch_max):
    @functools.partial(
        pl.kernel,
        out_type=jax.ShapeDtypeStruct((NW, n_pad), jnp.float32),
        mesh=_mesh,
        compiler_params=_sc_params,
        scratch_types=[
            pltpu.VMEM((n_pad,), jnp.float32),
            pltpu.VMEM((ch_max, C), jnp.int32),
        ],
    )
    def deg_kernel(pk_hbm, out_hbm, hist, pkbuf):
        c = lax.axis_index("c")
        s = lax.axis_index("s")
        wid = s * NC + c

        def zero_body(i, _):
            hist[pl.ds(i * 16, 16)] = jnp.zeros((16,), jnp.float32)
            return 0

        lax.fori_loop(0, n_pad // 16, zero_body, 0)
        pltpu.sync_copy(pk_hbm.at[wid], pkbuf)

        ones = jnp.full((16,), 1.0, jnp.float32)

        def body(k, _):
            for j in range(C // 16):
                v = pkbuf[k, pl.ds(j * 16, 16)]
                dst16 = lax.shift_right_logical(v, SHIFT)
                plsc.addupdate_scatter(hist, [dst16], ones)
            return 0

        lax.fori_loop(0, ch_max, body, 0)
        pltpu.sync_copy(hist, out_hbm.at[wid])

    return deg_kernel


def _make_prop_kernel(n, n_pad, ch_max, d, rows_per):
    @functools.partial(
        pl.kernel,
        out_type=jax.ShapeDtypeStruct((NC, n, d), jnp.float32),
        mesh=_mesh,
        compiler_params=_sc_params,
        scratch_types=[
            pltpu.VMEM_SHARED((n_pad, d), jnp.float32),
            pltpu.VMEM((ch_max, C), jnp.int32),
            pltpu.VMEM((NBUF, C), jnp.int32),
            pltpu.VMEM((NBUF, C), jnp.int32),
            tuple(pltpu.VMEM((C, d), jnp.float32) for _ in range(NBUF)),
            pltpu.SemaphoreType.DMA((NBUF,)),
        ],
    )
    def prop_kernel(p_hbm, pk_hbm, out_hbm, acc, pkbuf, sring, dring, rows,
                    gsem):
        c = lax.axis_index("c")
        s = lax.axis_index("s")
        wid = s * NC + c
        nch = lax.select(c == 0, CH0, CH1)

        tail = n - NS * rows_per
        pltpu.sync_copy(
            p_hbm.at[pl.ds(s * rows_per, rows_per)],
            acc.at[pl.ds(s * rows_per, rows_per)],
        )
        @pl.when(s == 0)
        def _():
            pltpu.sync_copy(
                p_hbm.at[pl.ds(NS * rows_per, tail)],
                acc.at[pl.ds(NS * rows_per, tail)],
            )
        pltpu.sync_copy(pk_hbm.at[wid], pkbuf)

        def unpack(k, slot):
            for j in range(C // 16):
                v = pkbuf[k, pl.ds(j * 16, 16)]
                sring[slot, pl.ds(j * 16, 16)] = v & ((1 << SHIFT) - 1)
                dring[slot, pl.ds(j * 16, 16)] = lax.shift_right_logical(v, SHIFT)

        def fire_gather(k, b):
            pltpu.async_copy(p_hbm.at[sring.at[b]], rows[b], gsem.at[b])

        def wait_gather(k, b):
            pltpu.make_async_copy(p_hbm.at[sring.at[b]], rows[b], gsem.at[b]).wait()

        for b in range(NBUF - 1):
            unpack(b, b)
            fire_gather(b, b)
        plsc.subcore_barrier()

        def body(g, _):
            for b in range(NBUF):
                k = g * NBUF + b
                @pl.when(k + NBUF - 1 < nch)
                def _():
                    unpack(k + NBUF - 1, (b + NBUF - 1) % NBUF)
                    fire_gather(k + NBUF - 1, (b + NBUF - 1) % NBUF)

                wait_gather(k, b)
                pltpu.sync_copy(rows[b], acc.at[dring.at[b]], add=True)
            return 0

        lax.fori_loop(0, nch // NBUF, body, 0)
        plsc.subcore_barrier()
        pltpu.sync_copy(
            acc.at[pl.ds(s * rows_per, rows_per)],
            out_hbm.at[c, pl.ds(s * rows_per, rows_per)],
        )
        @pl.when(s == 0)
        def _():
            pltpu.sync_copy(
                acc.at[pl.ds(NS * rows_per, tail)],
                out_hbm.at[c, pl.ds(NS * rows_per, tail)],
            )

    return prop_kernel


def _deg_reduce_body(hist_ref, dis_ref):
    deg = 1.0 + jnp.sum(hist_ref[...], axis=0, keepdims=True)
    dis_ref[...] = lax.rsqrt(deg)


def _t0_body(x_ref, dis_ref, w_ref, p_ref):
    q = jnp.dot(x_ref[...], w_ref[...], preferred_element_type=jnp.float32)
    p_ref[...] = q * dis_ref[...]


def _t_mid_body(acc_ref, p_ref, dis_ref, b_ref, w_ref, o_ref):
    dis = dis_ref[...]
    a = acc_ref[...]
    h = (a[0] + a[1] - p_ref[...]) * dis + b_ref[...]
    h = jnp.maximum(h, 0.0)
    o_ref[...] = jnp.dot(h, w_ref[...], preferred_element_type=jnp.float32) * dis


def _t_final_body(acc_ref, p_ref, dis_ref, b_ref, o_ref):
    a = acc_ref[...]
    o_ref[...] = (a[0] + a[1] - p_ref[...]) * dis_ref[...] + b_ref[...]


def kernel(x, edge_index, batch, W1, b1, W2, b2, W3, b3):
    n, d_in = x.shape
    e = edge_index.shape[1]
    d_hid = W1.shape[1]
    n_cls = W3.shape[1]

    ch_max = max(CH0, CH1)
    n_pad = -(-(n + 1) // NS) * NS
    rows_per = (n // NS) // 8 * 8

    src = edge_index[0]
    dst = edge_index[1]
    pad_val = jnp.int32(n << SHIFT)
    packed = src + (dst << SHIFT)

    cap1 = NS * CH1 * C
    cap0 = NS * CH0 * C
    r1 = cap1
    pk1 = packed[:r1].reshape(NS, CH1, C)
    pk0 = jnp.pad(packed[r1:], (0, cap0 - (e - r1)),
                  constant_values=pad_val).reshape(NS, CH0, C)
    pk0 = jnp.pad(pk0, ((0, 0), (0, ch_max - CH0), (0, 0)),
                  constant_values=pad_val)
    pk1 = jnp.pad(pk1, ((0, 0), (0, ch_max - CH1), (0, 0)),
                  constant_values=pad_val)
    pk = jnp.stack([pk0, pk1], axis=1).reshape(NW, ch_max, C)

    hist = _make_deg_kernel(n_pad, ch_max)(pk)

    bn = 400
    grid = n // bn
    f32 = jnp.float32

    dis_full = pl.pallas_call(
        _deg_reduce_body,
        grid=(1,),
        in_specs=[pl.BlockSpec((NW, n_pad), lambda j: (0, 0))],
        out_specs=pl.BlockSpec((1, n_pad), lambda j: (0, 0)),
        out_shape=jax.ShapeDtypeStruct((1, n_pad), f32),
    )(hist)
    dis = dis_full[0, :n].reshape(n, 1)

    p1 = pl.pallas_call(
        _t0_body,
        grid=(grid,),
        in_specs=[
            pl.BlockSpec((bn, d_in), lambda j: (j, 0)),
            pl.BlockSpec((bn, 1), lambda j: (j, 0)),
            pl.BlockSpec((d_in, d_hid), lambda j: (0, 0)),
        ],
        out_specs=pl.BlockSpec((bn, d_hid), lambda j: (j, 0)),
        out_shape=jax.ShapeDtypeStruct((n, d_hid), f32),
    )(x, dis, W1)

    prop_h = _make_prop_kernel(n, n_pad, ch_max, d_hid, rows_per)
    prop_c = _make_prop_kernel(n, n_pad, ch_max, n_cls, rows_per)

    def t_mid(acc, p, dis, b, w, d_out):
        return pl.pallas_call(
            _t_mid_body,
            grid=(grid,),
            in_specs=[
                pl.BlockSpec((NC, bn, d_hid), lambda j: (0, j, 0)),
                pl.BlockSpec((bn, d_hid), lambda j: (j, 0)),
                pl.BlockSpec((bn, 1), lambda j: (j, 0)),
                pl.BlockSpec((1, d_hid), lambda j: (0, 0)),
                pl.BlockSpec((d_hid, d_out), lambda j: (0, 0)),
            ],
            out_specs=pl.BlockSpec((bn, d_out), lambda j: (j, 0)),
            out_shape=jax.ShapeDtypeStruct((n, d_out), f32),
        )(acc, p, dis, b.reshape(1, -1), w)

    a1 = prop_h(p1, pk)
    p2 = t_mid(a1, p1, dis, b1, W2, d_hid)
    a2 = prop_h(p2, pk)
    p3 = t_mid(a2, p2, dis, b2, W3, n_cls)
    a3 = prop_c(p3, pk)

    out = pl.pallas_call(
        _t_final_body,
        grid=(grid,),
        in_specs=[
            pl.BlockSpec((NC, bn, n_cls), lambda j: (0, j, 0)),
            pl.BlockSpec((bn, n_cls), lambda j: (j, 0)),
            pl.BlockSpec((bn, 1), lambda j: (j, 0)),
            pl.BlockSpec((1, n_cls), lambda j: (0, 0)),
        ],
        out_specs=pl.BlockSpec((bn, n_cls), lambda j: (j, 0)),
        out_shape=jax.ShapeDtypeStruct((n, n_cls), f32),
    )(a3, p3, dis, b3.reshape(1, -1))
    return out


# speedup vs baseline: 1.1697x; 1.0773x over previous
import functools

import jax
import jax.numpy as jnp
from jax import lax
from jax.experimental import pallas as pl
from jax.experimental.pallas import tpu as pltpu
from jax.experimental.pallas import tpu_sc as plsc

NC = 2
NS = 16
NW = NC * NS
C = 96
NBUF = 3
CH0 = 84
CH1 = 126
SHIFT = 14

_mesh = plsc.VectorSubcoreMesh(
    core_axis_name="c", subcore_axis_name="s", num_cores=NC, num_subcores=NS
)
_sc_params = pltpu.CompilerParams(
    needs_layout_passes=False, use_tc_tiling_on_sc=False
)


def _make_deg_kernel(n_pad, ch_max):
    @functools.partial(
        pl.kernel,
        out_type=jax.ShapeDtypeStruct((NW, n_pad), jnp.float32),
        mesh=_mesh,
        compiler_params=_sc_params,
        scratch_types=[
            pltpu.VMEM((n_pad,), jnp.float32),
            pltpu.VMEM((ch_max, C), jnp.int32),
        ],
    )
    def deg_kernel(pk_hbm, out_hbm, hist, pkbuf):
        c = lax.axis_index("c")
        s = lax.axis_index("s")
        wid = s * NC + c

        def zero_body(i, _):
            hist[pl.ds(i * 16, 16)] = jnp.zeros((16,), jnp.float32)
            return 0

        lax.fori_loop(0, n_pad // 16, zero_body, 0)
        pltpu.sync_copy(pk_hbm.at[wid], pkbuf)

        ones = jnp.full((16,), 1.0, jnp.float32)

        def body(k, _):
            for j in range(C // 16):
                v = pkbuf[k, pl.ds(j * 16, 16)]
                dst16 = lax.shift_right_logical(v, SHIFT)
                plsc.addupdate_scatter(hist, [dst16], ones)
            return 0

        lax.fori_loop(0, ch_max, body, 0)
        pltpu.sync_copy(hist, out_hbm.at[wid])

    return deg_kernel


def _make_prop_kernel(n, n_pad, ch_max, d, rows_per):
    @functools.partial(
        pl.kernel,
        out_type=jax.ShapeDtypeStruct((NC, n, d), jnp.float32),
        mesh=_mesh,
        compiler_params=_sc_params,
        scratch_types=[
            pltpu.VMEM_SHARED((n_pad, d), jnp.float32),
            pltpu.VMEM((ch_max, C), jnp.int32),
            pltpu.VMEM((NBUF, C), jnp.int32),
            pltpu.VMEM((NBUF, C), jnp.int32),
            tuple(pltpu.VMEM((C, d), jnp.float32) for _ in range(NBUF)),
            pltpu.SemaphoreType.DMA((NBUF,)),
        ],
    )
    def prop_kernel(p_hbm, pk_hbm, out_hbm, acc, pkbuf, sring, dring, rows,
                    gsem):
        c = lax.axis_index("c")
        s = lax.axis_index("s")
        wid = s * NC + c
        nch = lax.select(c == 0, CH0, CH1)

        tail = n - NS * rows_per
        pltpu.sync_copy(
            p_hbm.at[pl.ds(s * rows_per, rows_per)],
            acc.at[pl.ds(s * rows_per, rows_per)],
        )
        @pl.when(s == 0)
        def _():
            pltpu.sync_copy(
                p_hbm.at[pl.ds(NS * rows_per, tail)],
                acc.at[pl.ds(NS * rows_per, tail)],
            )
        pltpu.sync_copy(pk_hbm.at[wid], pkbuf)

        def unpack(k, slot):
            for j in range(C // 16):
                v = pkbuf[k, pl.ds(j * 16, 16)]
                sring[slot, pl.ds(j * 16, 16)] = v & ((1 << SHIFT) - 1)
                dring[slot, pl.ds(j * 16, 16)] = lax.shift_right_logical(v, SHIFT)

        def fire_gather(k, b):
            pltpu.async_copy(p_hbm.at[sring.at[b]], rows[b], gsem.at[b])

        def wait_gather(k, b):
            pltpu.make_async_copy(p_hbm.at[sring.at[b]], rows[b], gsem.at[b]).wait()

        for b in range(NBUF - 1):
            unpack(b, b)
            fire_gather(b, b)
        plsc.subcore_barrier()

        def body(g, _):
            for b in range(NBUF):
                k = g * NBUF + b
                @pl.when(k + NBUF - 1 < nch)
                def _():
                    unpack(k + NBUF - 1, (b + NBUF - 1) % NBUF)
                    fire_gather(k + NBUF - 1, (b + NBUF - 1) % NBUF)

                wait_gather(k, b)
                pltpu.sync_copy(rows[b], acc.at[dring.at[b]], add=True)
            return 0

        lax.fori_loop(0, nch // NBUF, body, 0)
        plsc.subcore_barrier()
        pltpu.sync_copy(
            acc.at[pl.ds(s * rows_per, rows_per)],
            out_hbm.at[c, pl.ds(s * rows_per, rows_per)],
        )
        @pl.when(s == 0)
        def _():
            pltpu.sync_copy(
                acc.at[pl.ds(NS * rows_per, tail)],
                out_hbm.at[c, pl.ds(NS * rows_per, tail)],
            )

    return prop_kernel


def _deg_reduce_body(hist_ref, dis_ref):
    deg = 1.0 + jnp.sum(hist_ref[...], axis=0, keepdims=True)
    dis_ref[...] = lax.rsqrt(deg)


def _t0_body(x_ref, dis_ref, w_ref, p_ref):
    q = jnp.dot(x_ref[...], w_ref[...], preferred_element_type=jnp.float32)
    p_ref[...] = q * dis_ref[...]


def _t_mid_body(acc_ref, p_ref, dis_ref, b_ref, w_ref, o_ref):
    dis = dis_ref[...]
    a = acc_ref[...]
    h = (a[0] + a[1] - p_ref[...]) * dis + b_ref[...]
    h = jnp.maximum(h, 0.0)
    o_ref[...] = jnp.dot(h, w_ref[...], preferred_element_type=jnp.float32) * dis


def _t_final_body(acc_ref, p_ref, dis_ref, b_ref, o_ref):
    a = acc_ref[...]
    o_ref[...] = (a[0] + a[1] - p_ref[...]) * dis_ref[...] + b_ref[...]


def kernel(x, edge_index, batch, W1, b1, W2, b2, W3, b3):
    n, d_in = x.shape
    e = edge_index.shape[1]
    d_hid = W1.shape[1]
    n_cls = W3.shape[1]

    ch_max = max(CH0, CH1)
    n_pad = -(-(n + 1) // NS) * NS
    rows_per = (n // NS) // 8 * 8

    src = edge_index[0]
    dst = edge_index[1]
    pad_val = jnp.int32(n << SHIFT)
    packed = src + (dst << SHIFT)

    cap1 = NS * CH1 * C
    cap0 = NS * CH0 * C
    r1 = cap1
    pk1 = packed[:r1].reshape(NS, CH1, C)
    pk0 = jnp.pad(packed[r1:], (0, cap0 - (e - r1)),
                  constant_values=pad_val).reshape(NS, CH0, C)
    pk0 = jnp.pad(pk0, ((0, 0), (0, ch_max - CH0), (0, 0)),
                  constant_values=pad_val)
    pk1 = jnp.pad(pk1, ((0, 0), (0, ch_max - CH1), (0, 0)),
                  constant_values=pad_val)
    pk = jnp.stack([pk0, pk1], axis=1).reshape(NW, ch_max, C)

    hist = _make_deg_kernel(n_pad, ch_max)(pk)

    bn = 400
    grid = n // bn
    f32 = jnp.float32

    dis_full = pl.pallas_call(
        _deg_reduce_body,
        grid=(1,),
        in_specs=[pl.BlockSpec((NW, n_pad), lambda j: (0, 0))],
        out_specs=pl.BlockSpec((1, n_pad), lambda j: (0, 0)),
        out_shape=jax.ShapeDtypeStruct((1, n_pad), f32),
    )(hist)
    dis = dis_full[0, :n].reshape(n, 1)

    p1 = pl.pallas_call(
        _t0_body,
        grid=(grid,),
        in_specs=[
            pl.BlockSpec((bn, d_in), lambda j: (j, 0)),
            pl.BlockSpec((bn, 1), lambda j: (j, 0)),
            pl.BlockSpec((d_in, d_hid), lambda j: (0, 0)),
        ],
        out_specs=pl.BlockSpec((bn, d_hid), lambda j: (j, 0)),
        out_shape=jax.ShapeDtypeStruct((n, d_hid), f32),
    )(x, dis, W1)

    prop_h = _make_prop_kernel(n, n_pad, ch_max, d_hid, rows_per)
    prop_c = _make_prop_kernel(n, n_pad, ch_max, n_cls, rows_per)

    def t_mid(acc, p, dis, b, w, d_out):
        return pl.pallas_call(
            _t_mid_body,
            grid=(grid,),
            in_specs=[
                pl.BlockSpec((NC, bn, d_hid), lambda j: (0, j, 0)),
                pl.BlockSpec((bn, d_hid), lambda j: (j, 0)),
                pl.BlockSpec((bn, 1), lambda j: (j, 0)),
                pl.BlockSpec((1, d_hid), lambda j: (0, 0)),
                pl.BlockSpec((d_hid, d_out), lambda j: (0, 0)),
            ],
            out_specs=pl.BlockSpec((bn, d_out), lambda j: (j, 0)),
            out_shape=jax.ShapeDtypeStruct((n, d_out), f32),
        )(acc, p, dis, b.reshape(1, -1), w)

    a1 = prop_h(p1, pk)
    p2 = t_mid(a1, p1, dis, b1, W2, d_hid)
    a2 = prop_h(p2, pk)
    p3 = t_mid(a2, p2, dis, b2, W3, n_cls)
    a3 = prop_c(p3, pk)

    out = pl.pallas_call(
        _t_final_body,
        grid=(grid,),
        in_specs=[
            pl.BlockSpec((NC, bn, n_cls), lambda j: (0, j, 0)),
            pl.BlockSpec((bn, n_cls), lambda j: (j, 0)),
            pl.BlockSpec((bn, 1), lambda j: (j, 0)),
            pl.BlockSpec((1, n_cls), lambda j: (0, 0)),
        ],
        out_specs=pl.BlockSpec((bn, n_cls), lambda j: (j, 0)),
        out_shape=jax.ShapeDtypeStruct((n, n_cls), f32),
    )(a3, p3, dis, b3.reshape(1, -1))
    return out
